# baseline (speedup 1.0000x reference)
import functools

import jax
import jax.numpy as jnp
from jax import lax
from jax.experimental import pallas as pl
from jax.experimental.pallas import tpu as pltpu
from jax.experimental.pallas import tpu_sc as plsc

N = 10000
D = 128
NC = 2
NS = 16
NW = NC * NS
CHUNK = 128

ACC_ROWS = 10112
ACC_PER_TILE = ACC_ROWS // NS
EPS = 1e-5
PAD = -1

_MESH = plsc.VectorSubcoreMesh(core_axis_name="c", subcore_axis_name="s")


def _fill_pad_row(idx_v, row):
  for k in range(CHUNK // 16):
    idx_v[row, pl.ds(k * 16, 16)] = jnp.full((16,), PAD, jnp.int32)


def _stage_tail(ef_hbm, base, nfull, lc, idx_v):
  _fill_pad_row(idx_v, 2)
  if lc:
    pltpu.sync_copy(ef_hbm.at[pl.ds(base + nfull * CHUNK, lc)],
                    idx_v.at[2, pl.ds(0, lc)])


def _deg_body(etile, edges, dst_hbm, zeros_hbm, ones_hbm, deg_out,
              idx_v, ones_v, stage_v, semd, acc_sh):
  nfull, lc = divmod(etile, CHUNK)
  nch = nfull + (1 if lc else 0)
  c = lax.axis_index("c")
  s = lax.axis_index("s")
  wid = c * NS + s
  base = edges + wid * etile

  def row(j):
    return 2 if (lc and j == nch - 1) else j % 2

  _stage_tail(dst_hbm, base, nfull, lc, idx_v)
  for j in range(min(2, nfull)):
    pltpu.sync_copy(dst_hbm.at[pl.ds(base + j * CHUNK, CHUNK)], idx_v.at[j])
  pltpu.sync_copy(ones_hbm, ones_v)
  pltpu.sync_copy(zeros_hbm, stage_v)
  pltpu.sync_copy(stage_v, acc_sh.at[pl.ds(s * ACC_PER_TILE, ACC_PER_TILE)])
  plsc.subcore_barrier()

  pending = []
  for j in range(nch):
    if pending and pending[0] == j:
      pending.pop(0)
      dcp.wait()
    pltpu.sync_copy(ones_v, acc_sh.at[plsc.Indices(idx_v.at[row(j)], PAD)],
                    add=True)
    if j + 2 < nfull:
      dcp = pltpu.async_copy(
          dst_hbm.at[pl.ds(base + (j + 2) * CHUNK, CHUNK)],
          idx_v.at[(j + 2) % 2], semd)
      pending.append(j + 2)
  plsc.subcore_barrier()
  pltpu.sync_copy(acc_sh.at[pl.ds(s * ACC_PER_TILE, ACC_PER_TILE)], stage_v)
  pltpu.sync_copy(stage_v, deg_out.at[c, s, 0])


def _spmm_body(etile, edges, xs_hbm, ef_hbm, zeros_hbm, pout_hbm,
               sidx_v, didx_v, rows_v, semi, semd, semg0, semg1, acc_sh):
  nfull, lc = divmod(etile, CHUNK)
  nch = nfull + (1 if lc else 0)
  c = lax.axis_index("c")
  s = lax.axis_index("s")
  wid = c * NS + s
  sbase = wid * etile
  dbase = edges + wid * etile

  def row(j):
    return 2 if (lc and j == nch - 1) else j % 2

  _stage_tail(ef_hbm, sbase, nfull, lc, sidx_v)
  _stage_tail(ef_hbm, dbase, nfull, lc, didx_v)
  for j in range(min(2, nfull)):
    pltpu.sync_copy(ef_hbm.at[pl.ds(sbase + j * CHUNK, CHUNK)], sidx_v.at[j])
    pltpu.sync_copy(ef_hbm.at[pl.ds(dbase + j * CHUNK, CHUNK)], didx_v.at[j])

  pltpu.sync_copy(zeros_hbm, rows_v.at[0])
  abase = s * ACC_PER_TILE
  for k in range(ACC_PER_TILE // CHUNK):
    pltpu.sync_copy(rows_v.at[0], acc_sh.at[pl.ds(abase + k * CHUNK, CHUNK)])
  rem = ACC_PER_TILE % CHUNK
  if rem:
    pltpu.sync_copy(
        rows_v.at[0, pl.ds(0, rem)],
        acc_sh.at[pl.ds(abase + ACC_PER_TILE - rem, rem)])
  plsc.subcore_barrier()

  semg = (semg0, semg1)
  spend, dpend = [], []
  cp = pltpu.async_copy(xs_hbm.at[plsc.Indices(sidx_v.at[row(0)], PAD)],
                        rows_v.at[0], semg[0])
  for j in range(nch):
    if j + 1 < nch:
      if spend and spend[0] == j + 1:
        spend.pop(0)
        icp.wait()
      nxt = pltpu.async_copy(
          xs_hbm.at[plsc.Indices(sidx_v.at[row(j + 1)], PAD)],
          rows_v.at[(j + 1) % 2], semg[(j + 1) % 2])
    cp.wait()
    if j + 2 < nfull:
      icp = pltpu.async_copy(ef_hbm.at[pl.ds(sbase + (j + 2) * CHUNK, CHUNK)],
                             sidx_v.at[(j + 2) % 2], semi)
      spend.append(j + 2)
    if dpend and dpend[0] == j:
      dpend.pop(0)
      dcp.wait()
    pltpu.sync_copy(rows_v.at[j % 2],
                    acc_sh.at[plsc.Indices(didx_v.at[row(j)], PAD)], add=True)
    if j + 2 < nfull:
      dcp = pltpu.async_copy(ef_hbm.at[pl.ds(dbase + (j + 2) * CHUNK, CHUNK)],
                             didx_v.at[(j + 2) % 2], semd)
      dpend.append(j + 2)
    if j + 1 < nch:
      cp = nxt
  plsc.subcore_barrier()

  for k in range(ACC_PER_TILE // CHUNK):
    r0 = abase + k * CHUNK
    pltpu.sync_copy(acc_sh.at[pl.ds(r0, CHUNK)], rows_v.at[0])
    pltpu.sync_copy(rows_v.at[0], pout_hbm.at[c, pl.ds(r0, CHUNK)])
  if rem:
    r0 = abase + ACC_PER_TILE - rem
    pltpu.sync_copy(acc_sh.at[pl.ds(r0, rem)], rows_v.at[0, pl.ds(0, rem)])
    pltpu.sync_copy(rows_v.at[0, pl.ds(0, rem)],
                    pout_hbm.at[c, pl.ds(r0, rem)])


def _make_deg_kernel(etile, edges):
  return pl.kernel(
      functools.partial(_deg_body, etile, edges),
      out_type=jax.ShapeDtypeStruct((NC, NS, 1, ACC_PER_TILE), jnp.float32),
      mesh=_MESH,
      scratch_types=[
          pltpu.VMEM((3, CHUNK), jnp.int32),
          pltpu.VMEM((CHUNK,), jnp.float32),
          pltpu.VMEM((ACC_PER_TILE,), jnp.float32),
          pltpu.SemaphoreType.DMA,
          pltpu.VMEM_SHARED((ACC_ROWS,), jnp.float32),
      ],
  )


def _make_spmm_kernel(etile, edges):
  return pl.kernel(
      functools.partial(_spmm_body, etile, edges),
      out_type=jax.ShapeDtypeStruct((NC, ACC_ROWS, D), jnp.float32),
      mesh=_MESH,
      scratch_types=[
          pltpu.VMEM((3, CHUNK), jnp.int32),
          pltpu.VMEM((3, CHUNK), jnp.int32),
          pltpu.VMEM((2, CHUNK, D), jnp.float32),
          pltpu.SemaphoreType.DMA,
          pltpu.SemaphoreType.DMA,
          pltpu.SemaphoreType.DMA,
          pltpu.SemaphoreType.DMA,
          pltpu.VMEM_SHARED((ACC_ROWS, D), jnp.float32),
      ],
  )


def _bn_xw(h, g, be, w, dinv):
  mean = jnp.mean(h, axis=0, keepdims=True)
  dlt = h - mean
  var = jnp.mean(dlt * dlt, axis=0, keepdims=True)
  hb = g * dlt * lax.rsqrt(var + EPS) + be
  return dinv * jnp.dot(hb, w, preferred_element_type=jnp.float32)


def _tc_front_body(x_ref, deg_ref, w_ref, g_ref, be_ref, xs_ref, dinv_ref):
  dinv_row = lax.rsqrt(deg_ref[0:1, :] + deg_ref[1:2, :] + 1.0)
  dinv_ref[...] = dinv_row
  dinv = jnp.transpose(dinv_row[:, :N])
  xs_ref[...] = _bn_xw(x_ref[...], g_ref[...], be_ref[...], w_ref[...],
                       dinv)


def _tc_mid_body(p_ref, xs_ref, dinv_ref, b_ref, w_ref, g_ref, be_ref,
                 h_ref, xsn_ref):
  dinv = jnp.transpose(dinv_ref[:, :N])
  ssum = p_ref[0, :N] + p_ref[1, :N] + xs_ref[...]
  h = jnp.maximum(dinv * ssum + b_ref[...], 0.0)
  h_ref[...] = h
  xsn_ref[...] = _bn_xw(h, g_ref[...], be_ref[...], w_ref[...], dinv)


def _tc_last_body(p_ref, xs_ref, dinv_ref, b_ref, h1_ref, h2_ref, out_ref):
  dinv = jnp.transpose(dinv_ref[:, :N])
  ssum = p_ref[0, :N] + p_ref[1, :N] + xs_ref[...]
  h3 = jnp.maximum(dinv * ssum + b_ref[...], 0.0)
  out_ref[...] = jnp.concatenate([h1_ref[...], h2_ref[...], h3], axis=-1)


_tc_front = pl.pallas_call(
    _tc_front_body,
    out_shape=(
        jax.ShapeDtypeStruct((N, D), jnp.float32),
        jax.ShapeDtypeStruct((1, ACC_ROWS), jnp.float32),
    ),
)

_tc_mid = pl.pallas_call(
    _tc_mid_body,
    out_shape=(
        jax.ShapeDtypeStruct((N, D), jnp.float32),
        jax.ShapeDtypeStruct((N, D), jnp.float32),
    ),
)

_tc_last = pl.pallas_call(
    _tc_last_body,
    out_shape=jax.ShapeDtypeStruct((N, 3 * D), jnp.float32),
)


@jax.jit
def _run(x, edge_index, W0, b0, g0, be0, W1, b1, g1, be1, W2, b2, g2, be2):
  e = edge_index.shape[1]
  etile = -(-e // (NW * 8)) * 8
  ep = etile * NW
  if ep != e:
    edge_index = jnp.concatenate(
        [edge_index, jnp.full((2, ep - e), PAD, jnp.int32)], axis=1)
  ef = edge_index.reshape(2 * ep)

  zeros_deg = jnp.zeros((ACC_PER_TILE,), jnp.float32)
  zeros_blk = jnp.zeros((CHUNK, D), jnp.float32)
  ones_row = jnp.ones((CHUNK,), jnp.float32)

  deg_p = _make_deg_kernel(etile, ep)(ef, zeros_deg, ones_row)
  deg_p = deg_p.reshape(NC, ACC_ROWS)

  spmm = _make_spmm_kernel(etile, ep)
  g0r, be0r, b0r = g0.reshape(1, D), be0.reshape(1, D), b0.reshape(1, D)
  g1r, be1r, b1r = g1.reshape(1, D), be1.reshape(1, D), b1.reshape(1, D)
  g2r, be2r, b2r = g2.reshape(1, D), be2.reshape(1, D), b2.reshape(1, D)

  xs0, dinv = _tc_front(x, deg_p, W0, g0r, be0r)
  p0 = spmm(xs0, ef, zeros_blk)
  h1, xs1 = _tc_mid(p0, xs0, dinv, b0r, W1, g1r, be1r)
  p1 = spmm(xs1, ef, zeros_blk)
  h2, xs2 = _tc_mid(p1, xs1, dinv, b1r, W2, g2r, be2r)
  p2 = spmm(xs2, ef, zeros_blk)
  return _tc_last(p2, xs2, dinv, b2r, h1, h2)


def kernel(x, edge_index, W0, b0, g0, be0, W1, b1, g1, be1, W2, b2, g2, be2):
  return _run(x, edge_index, W0, b0, g0, be0, W1, b1, g1, be1, W2, b2, g2,
              be2)


# speedup vs baseline: 1.0481x; 1.0481x over previous
import functools

import jax
import jax.numpy as jnp
from jax import lax
from jax.experimental import pallas as pl
from jax.experimental.pallas import tpu as pltpu
from jax.experimental.pallas import tpu_sc as plsc

N = 10000
D = 128
NC = 2
NS = 16
NW = NC * NS
CHUNK = 128
GRP = 8

ACC_ROWS = 10112
ACC_PER_TILE = ACC_ROWS // NS
EPS = 1e-5
PAD = -1

_MESH = plsc.VectorSubcoreMesh(core_axis_name="c", subcore_axis_name="s")


def _deg_body(nchunks, dst_hbm, zeros_hbm, ones_hbm, deg_out, idx_v, ones_v,
              stage_v, acc_sh):
  c = lax.axis_index("c")
  s = lax.axis_index("s")
  wid = c * NS + s
  pltpu.sync_copy(dst_hbm.at[wid], idx_v)
  pltpu.sync_copy(ones_hbm, ones_v)
  pltpu.sync_copy(zeros_hbm, stage_v)
  pltpu.sync_copy(stage_v, acc_sh.at[pl.ds(s * ACC_PER_TILE, ACC_PER_TILE)])
  plsc.subcore_barrier()
  for j in range(nchunks):
    pltpu.sync_copy(ones_v, acc_sh.at[plsc.Indices(idx_v.at[j], PAD)],
                    add=True)
  plsc.subcore_barrier()
  pltpu.sync_copy(acc_sh.at[pl.ds(s * ACC_PER_TILE, ACC_PER_TILE)], stage_v)
  pltpu.sync_copy(stage_v, deg_out.at[c, s, 0])


def _spmm_body(etile, nchunks, xs_hbm, ef_hbm, dst_hbm, zeros_hbm, pout_hbm,
               sidx0_v, sidx1_v, didx_v, rows_v, semi, semg0, semg1, acc_sh):
  sidx = (sidx0_v, sidx1_v)
  ngrp = -(-nchunks // GRP)
  c = lax.axis_index("c")
  s = lax.axis_index("s")
  wid = c * NS + s
  sbase = wid * etile

  pltpu.sync_copy(dst_hbm.at[wid], didx_v)
  pltpu.sync_copy(ef_hbm.at[pl.ds(sbase, GRP * CHUNK)], sidx[0])

  pltpu.sync_copy(zeros_hbm, rows_v.at[0])
  abase = s * ACC_PER_TILE
  for k in range(ACC_PER_TILE // CHUNK):
    pltpu.sync_copy(rows_v.at[0], acc_sh.at[pl.ds(abase + k * CHUNK, CHUNK)])
  rem = ACC_PER_TILE % CHUNK
  if rem:
    pltpu.sync_copy(
        rows_v.at[0, pl.ds(0, rem)],
        acc_sh.at[pl.ds(abase + ACC_PER_TILE - rem, rem)])
  plsc.subcore_barrier()

  semg = (semg0, semg1)
  icp = None
  cp = pltpu.async_copy(
      xs_hbm.at[plsc.Indices(sidx[0].at[pl.ds(0, CHUNK)], PAD)],
      rows_v.at[0], semg[0])
  for j in range(nchunks):
    g, i = divmod(j, GRP)
    if i == 0 and g + 1 < ngrp:
      icp = pltpu.async_copy(
          ef_hbm.at[pl.ds(sbase + (g + 1) * GRP * CHUNK, GRP * CHUNK)],
          sidx[(g + 1) % 2], semi)
    if j + 1 < nchunks:
      g1, i1 = divmod(j + 1, GRP)
      if i1 == 0:
        icp.wait()
      nxt = pltpu.async_copy(
          xs_hbm.at[plsc.Indices(sidx[g1 % 2].at[pl.ds(i1 * CHUNK, CHUNK)],
                                 PAD)],
          rows_v.at[(j + 1) % 2], semg[(j + 1) % 2])
    cp.wait()
    pltpu.sync_copy(rows_v.at[j % 2],
                    acc_sh.at[plsc.Indices(didx_v.at[j], PAD)], add=True)
    if j + 1 < nchunks:
      cp = nxt
  plsc.subcore_barrier()

  for k in range(ACC_PER_TILE // CHUNK):
    r0 = abase + k * CHUNK
    pltpu.sync_copy(acc_sh.at[pl.ds(r0, CHUNK)], rows_v.at[0])
    pltpu.sync_copy(rows_v.at[0], pout_hbm.at[c, pl.ds(r0, CHUNK)])
  if rem:
    r0 = abase + ACC_PER_TILE - rem
    pltpu.sync_copy(acc_sh.at[pl.ds(r0, rem)], rows_v.at[0, pl.ds(0, rem)])
    pltpu.sync_copy(rows_v.at[0, pl.ds(0, rem)],
                    pout_hbm.at[c, pl.ds(r0, rem)])


def _make_deg_kernel(ncs):
  return pl.kernel(
      functools.partial(_deg_body, ncs),
      out_type=jax.ShapeDtypeStruct((NC, NS, 1, ACC_PER_TILE), jnp.float32),
      mesh=_MESH,
      scratch_types=[
          pltpu.VMEM((ncs, CHUNK), jnp.int32),
          pltpu.VMEM((CHUNK,), jnp.float32),
          pltpu.VMEM((ACC_PER_TILE,), jnp.float32),
          pltpu.VMEM_SHARED((ACC_ROWS,), jnp.float32),
      ],
  )


def _make_spmm_kernel(etile, ncs, nchunks):
  return pl.kernel(
      functools.partial(_spmm_body, etile, nchunks),
      out_type=jax.ShapeDtypeStruct((NC, ACC_ROWS, D), jnp.float32),
      mesh=_MESH,
      scratch_types=[
          pltpu.VMEM((GRP * CHUNK,), jnp.int32),
          pltpu.VMEM((GRP * CHUNK,), jnp.int32),
          pltpu.VMEM((ncs, CHUNK), jnp.int32),
          pltpu.VMEM((2, CHUNK, D), jnp.float32),
          pltpu.SemaphoreType.DMA,
          pltpu.SemaphoreType.DMA,
          pltpu.SemaphoreType.DMA,
          pltpu.VMEM_SHARED((ACC_ROWS, D), jnp.float32),
      ],
  )


def _bn_xw(h, g, be, w, dinv):
  mean = jnp.mean(h, axis=0, keepdims=True)
  dlt = h - mean
  var = jnp.mean(dlt * dlt, axis=0, keepdims=True)
  hb = g * dlt * lax.rsqrt(var + EPS) + be
  return dinv * jnp.dot(hb, w, preferred_element_type=jnp.float32)


def _tc_front_body(x_ref, deg_ref, w_ref, g_ref, be_ref, xs_ref, dinv_ref):
  dinv_row = lax.rsqrt(deg_ref[0:1, :] + deg_ref[1:2, :] + 1.0)
  dinv_ref[...] = dinv_row
  dinv = jnp.transpose(dinv_row[:, :N])
  xs_ref[...] = _bn_xw(x_ref[...], g_ref[...], be_ref[...], w_ref[...],
                       dinv)


def _tc_mid_body(p_ref, xs_ref, dinv_ref, b_ref, w_ref, g_ref, be_ref,
                 h_ref, xsn_ref):
  dinv = jnp.transpose(dinv_ref[:, :N])
  ssum = p_ref[0, :N] + p_ref[1, :N] + xs_ref[...]
  h = jnp.maximum(dinv * ssum + b_ref[...], 0.0)
  h_ref[...] = h
  xsn_ref[...] = _bn_xw(h, g_ref[...], be_ref[...], w_ref[...], dinv)


def _tc_last_body(p_ref, xs_ref, dinv_ref, b_ref, h1_ref, h2_ref, out_ref):
  dinv = jnp.transpose(dinv_ref[:, :N])
  ssum = p_ref[0, :N] + p_ref[1, :N] + xs_ref[...]
  h3 = jnp.maximum(dinv * ssum + b_ref[...], 0.0)
  out_ref[...] = jnp.concatenate([h1_ref[...], h2_ref[...], h3], axis=-1)


_tc_front = pl.pallas_call(
    _tc_front_body,
    out_shape=(
        jax.ShapeDtypeStruct((N, D), jnp.float32),
        jax.ShapeDtypeStruct((1, ACC_ROWS), jnp.float32),
    ),
)

_tc_mid = pl.pallas_call(
    _tc_mid_body,
    out_shape=(
        jax.ShapeDtypeStruct((N, D), jnp.float32),
        jax.ShapeDtypeStruct((N, D), jnp.float32),
    ),
)

_tc_last = pl.pallas_call(
    _tc_last_body,
    out_shape=jax.ShapeDtypeStruct((N, 3 * D), jnp.float32),
)


@jax.jit
def _run(x, edge_index, W0, b0, g0, be0, W1, b1, g1, be1, W2, b2, g2, be2):
  e = edge_index.shape[1]
  etile = -(-e // (NW * 8)) * 8
  ep = etile * NW
  if ep != e:
    edge_index = jnp.concatenate(
        [edge_index, jnp.full((2, ep - e), PAD, jnp.int32)], axis=1)
  ef = edge_index.reshape(2 * ep)

  nchunks = -(-etile // CHUNK)
  ncs = -(-nchunks // GRP) * GRP
  ppt = ncs * CHUNK - etile
  dflat = lax.slice(ef, (ep,), (2 * ep,)).reshape(NW, etile)
  dst_p = jnp.concatenate(
      [dflat, jnp.full((NW, ppt), PAD, jnp.int32)], axis=1
  ).reshape(NW, ncs, CHUNK)

  zeros_deg = jnp.zeros((ACC_PER_TILE,), jnp.float32)
  zeros_blk = jnp.zeros((CHUNK, D), jnp.float32)
  ones_row = jnp.ones((CHUNK,), jnp.float32)

  deg_p = _make_deg_kernel(ncs)(dst_p, zeros_deg, ones_row)
  deg_p = deg_p.reshape(NC, ACC_ROWS)

  spmm = _make_spmm_kernel(etile, ncs, nchunks)
  g0r, be0r, b0r = g0.reshape(1, D), be0.reshape(1, D), b0.reshape(1, D)
  g1r, be1r, b1r = g1.reshape(1, D), be1.reshape(1, D), b1.reshape(1, D)
  g2r, be2r, b2r = g2.reshape(1, D), be2.reshape(1, D), b2.reshape(1, D)

  xs0, dinv = _tc_front(x, deg_p, W0, g0r, be0r)
  p0 = spmm(xs0, ef, dst_p, zeros_blk)
  h1, xs1 = _tc_mid(p0, xs0, dinv, b0r, W1, g1r, be1r)
  p1 = spmm(xs1, ef, dst_p, zeros_blk)
  h2, xs2 = _tc_mid(p1, xs1, dinv, b1r, W2, g2r, be2r)
  p2 = spmm(xs2, ef, dst_p, zeros_blk)
  return _tc_last(p2, xs2, dinv, b2r, h1, h2)


def kernel(x, edge_index, W0, b0, g0, be0, W1, b1, g1, be1, W2, b2, g2, be2):
  return _run(x, edge_index, W0, b0, g0, be0, W1, b1, g1, be1, W2, b2, g2,
              be2)


# speedup vs baseline: 1.0670x; 1.0180x over previous
import functools

import jax
import jax.numpy as jnp
from jax import lax
from jax.experimental import pallas as pl
from jax.experimental.pallas import tpu as pltpu
from jax.experimental.pallas import tpu_sc as plsc

N = 10000
D = 128
NC = 2
NS = 16
NW = NC * NS
CHUNK = 128
GRP = 8

ACC_ROWS = 10112
ACC_PER_TILE = ACC_ROWS // NS
EPS = 1e-5
PAD = -1

_MESH = plsc.VectorSubcoreMesh(core_axis_name="c", subcore_axis_name="s")


def _deg_body(nchunks, dst_hbm, zeros_hbm, ones_hbm, deg_out, idx_v, ones_v,
              stage_v, acc_sh):
  c = lax.axis_index("c")
  s = lax.axis_index("s")
  wid = c * NS + s
  pltpu.sync_copy(dst_hbm.at[wid], idx_v)
  pltpu.sync_copy(ones_hbm, ones_v)
  pltpu.sync_copy(zeros_hbm, stage_v)
  pltpu.sync_copy(stage_v, acc_sh.at[pl.ds(s * ACC_PER_TILE, ACC_PER_TILE)])
  plsc.subcore_barrier()
  for j in range(nchunks):
    pltpu.sync_copy(ones_v, acc_sh.at[plsc.Indices(idx_v.at[j], PAD)],
                    add=True)
  plsc.subcore_barrier()
  pltpu.sync_copy(acc_sh.at[pl.ds(s * ACC_PER_TILE, ACC_PER_TILE)], stage_v)
  pltpu.sync_copy(stage_v, deg_out.at[c, s, 0])


def _spmm_body(etile, nchunks, xs_hbm, ef_hbm, dst_hbm, zeros_hbm, pout_hbm,
               sidx0_v, sidx1_v, didx_v, rows_v, semi, semg0, semg1, acc_sh):
  sidx = (sidx0_v, sidx1_v)
  ngrp = -(-nchunks // GRP)
  c = lax.axis_index("c")
  s = lax.axis_index("s")
  wid = c * NS + s
  sbase = wid * etile

  semg = (semg0, semg1)
  icp = pltpu.async_copy(ef_hbm.at[pl.ds(sbase, GRP * CHUNK)], sidx[0], semi)
  dcp = pltpu.async_copy(dst_hbm.at[wid], didx_v, semg0)

  pltpu.sync_copy(zeros_hbm, rows_v.at[0])
  abase = s * ACC_PER_TILE
  for k in range(ACC_PER_TILE // CHUNK):
    pltpu.sync_copy(rows_v.at[0], acc_sh.at[pl.ds(abase + k * CHUNK, CHUNK)])
  rem = ACC_PER_TILE % CHUNK
  if rem:
    pltpu.sync_copy(
        rows_v.at[0, pl.ds(0, rem)],
        acc_sh.at[pl.ds(abase + ACC_PER_TILE - rem, rem)])
  icp.wait()
  dcp.wait()

  cp = pltpu.async_copy(
      xs_hbm.at[plsc.Indices(sidx[0].at[pl.ds(0, CHUNK)], PAD)],
      rows_v.at[0], semg[0])
  plsc.subcore_barrier()

  for j in range(nchunks):
    g, i = divmod(j, GRP)
    if i == 0 and g + 1 < ngrp:
      icp = pltpu.async_copy(
          ef_hbm.at[pl.ds(sbase + (g + 1) * GRP * CHUNK, GRP * CHUNK)],
          sidx[(g + 1) % 2], semi)
    if j + 1 < nchunks:
      g1, i1 = divmod(j + 1, GRP)
      if i1 == 0:
        icp.wait()
      nxt = pltpu.async_copy(
          xs_hbm.at[plsc.Indices(sidx[g1 % 2].at[pl.ds(i1 * CHUNK, CHUNK)],
                                 PAD)],
          rows_v.at[(j + 1) % 2], semg[(j + 1) % 2])
    cp.wait()
    pltpu.sync_copy(rows_v.at[j % 2],
                    acc_sh.at[plsc.Indices(didx_v.at[j], PAD)], add=True)
    if j + 1 < nchunks:
      cp = nxt
  plsc.subcore_barrier()

  segs = [(abase + k * CHUNK, CHUNK) for k in range(ACC_PER_TILE // CHUNK)]
  if rem:
    segs.append((abase + ACC_PER_TILE - rem, rem))
  pend = [None, None]
  r0, ln = segs[0]
  pltpu.sync_copy(acc_sh.at[pl.ds(r0, ln)], rows_v.at[0, pl.ds(0, ln)])
  for k, (r0, ln) in enumerate(segs):
    b = k % 2
    pend[b] = pltpu.async_copy(rows_v.at[b, pl.ds(0, ln)],
                               pout_hbm.at[c, pl.ds(r0, ln)], semg[b])
    if k + 1 < len(segs):
      nb = (k + 1) % 2
      if pend[nb] is not None:
        pend[nb].wait()
        pend[nb] = None
      r1, ln1 = segs[k + 1]
      pltpu.sync_copy(acc_sh.at[pl.ds(r1, ln1)],
                      rows_v.at[nb, pl.ds(0, ln1)])
  for b in range(2):
    if pend[b] is not None:
      pend[b].wait()


def _make_deg_kernel(ncs):
  return pl.kernel(
      functools.partial(_deg_body, ncs),
      out_type=jax.ShapeDtypeStruct((NC, NS, 1, ACC_PER_TILE), jnp.float32),
      mesh=_MESH,
      scratch_types=[
          pltpu.VMEM((ncs, CHUNK), jnp.int32),
          pltpu.VMEM((CHUNK,), jnp.float32),
          pltpu.VMEM((ACC_PER_TILE,), jnp.float32),
          pltpu.VMEM_SHARED((ACC_ROWS,), jnp.float32),
      ],
  )


def _make_spmm_kernel(etile, ncs, nchunks):
  return pl.kernel(
      functools.partial(_spmm_body, etile, nchunks),
      out_type=jax.ShapeDtypeStruct((NC, ACC_ROWS, D), jnp.float32),
      mesh=_MESH,
      scratch_types=[
          pltpu.VMEM((GRP * CHUNK,), jnp.int32),
          pltpu.VMEM((GRP * CHUNK,), jnp.int32),
          pltpu.VMEM((ncs, CHUNK), jnp.int32),
          pltpu.VMEM((2, CHUNK, D), jnp.float32),
          pltpu.SemaphoreType.DMA,
          pltpu.SemaphoreType.DMA,
          pltpu.SemaphoreType.DMA,
          pltpu.VMEM_SHARED((ACC_ROWS, D), jnp.float32),
      ],
  )


def _bn_xw(h, g, be, w, dinv):
  mean = jnp.mean(h, axis=0, keepdims=True)
  dlt = h - mean
  var = jnp.mean(dlt * dlt, axis=0, keepdims=True)
  hb = g * dlt * lax.rsqrt(var + EPS) + be
  return dinv * jnp.dot(hb, w, preferred_element_type=jnp.float32)


def _tc_front_body(x_ref, deg_ref, w_ref, g_ref, be_ref, xs_ref, dinv_ref):
  dinv_row = lax.rsqrt(deg_ref[0:1, :] + deg_ref[1:2, :] + 1.0)
  dinv_ref[...] = dinv_row
  dinv = jnp.transpose(dinv_row[:, :N])
  xs_ref[...] = _bn_xw(x_ref[...], g_ref[...], be_ref[...], w_ref[...],
                       dinv)


def _tc_mid_body(p_ref, xs_ref, dinv_ref, b_ref, w_ref, g_ref, be_ref,
                 h_ref, xsn_ref):
  dinv = jnp.transpose(dinv_ref[:, :N])
  ssum = p_ref[0, :N] + p_ref[1, :N] + xs_ref[...]
  h = jnp.maximum(dinv * ssum + b_ref[...], 0.0)
  h_ref[...] = h
  xsn_ref[...] = _bn_xw(h, g_ref[...], be_ref[...], w_ref[...], dinv)


def _tc_last_body(p_ref, xs_ref, dinv_ref, b_ref, h1_ref, h2_ref, out_ref):
  dinv = jnp.transpose(dinv_ref[:, :N])
  ssum = p_ref[0, :N] + p_ref[1, :N] + xs_ref[...]
  h3 = jnp.maximum(dinv * ssum + b_ref[...], 0.0)
  out_ref[...] = jnp.concatenate([h1_ref[...], h2_ref[...], h3], axis=-1)


_tc_front = pl.pallas_call(
    _tc_front_body,
    out_shape=(
        jax.ShapeDtypeStruct((N, D), jnp.float32),
        jax.ShapeDtypeStruct((1, ACC_ROWS), jnp.float32),
    ),
)

_tc_mid = pl.pallas_call(
    _tc_mid_body,
    out_shape=(
        jax.ShapeDtypeStruct((N, D), jnp.float32),
        jax.ShapeDtypeStruct((N, D), jnp.float32),
    ),
)

_tc_last = pl.pallas_call(
    _tc_last_body,
    out_shape=jax.ShapeDtypeStruct((N, 3 * D), jnp.float32),
)


@jax.jit
def _run(x, edge_index, W0, b0, g0, be0, W1, b1, g1, be1, W2, b2, g2, be2):
  e = edge_index.shape[1]
  etile = -(-e // (NW * 8)) * 8
  ep = etile * NW
  if ep != e:
    edge_index = jnp.concatenate(
        [edge_index, jnp.full((2, ep - e), PAD, jnp.int32)], axis=1)
  ef = edge_index.reshape(2 * ep)

  nchunks = -(-etile // CHUNK)
  ncs = -(-nchunks // GRP) * GRP
  ppt = ncs * CHUNK - etile
  dflat = lax.slice(ef, (ep,), (2 * ep,)).reshape(NW, etile)
  dst_p = jnp.concatenate(
      [dflat, jnp.full((NW, ppt), PAD, jnp.int32)], axis=1
  ).reshape(NW, ncs, CHUNK)

  zeros_deg = jnp.zeros((ACC_PER_TILE,), jnp.float32)
  zeros_blk = jnp.zeros((CHUNK, D), jnp.float32)
  ones_row = jnp.ones((CHUNK,), jnp.float32)

  deg_p = _make_deg_kernel(ncs)(dst_p, zeros_deg, ones_row)
  deg_p = deg_p.reshape(NC, ACC_ROWS)

  spmm = _make_spmm_kernel(etile, ncs, nchunks)
  g0r, be0r, b0r = g0.reshape(1, D), be0.reshape(1, D), b0.reshape(1, D)
  g1r, be1r, b1r = g1.reshape(1, D), be1.reshape(1, D), b1.reshape(1, D)
  g2r, be2r, b2r = g2.reshape(1, D), be2.reshape(1, D), b2.reshape(1, D)

  xs0, dinv = _tc_front(x, deg_p, W0, g0r, be0r)
  p0 = spmm(xs0, ef, dst_p, zeros_blk)
  h1, xs1 = _tc_mid(p0, xs0, dinv, b0r, W1, g1r, be1r)
  p1 = spmm(xs1, ef, dst_p, zeros_blk)
  h2, xs2 = _tc_mid(p1, xs1, dinv, b1r, W2, g2r, be2r)
  p2 = spmm(xs2, ef, dst_p, zeros_blk)
  return _tc_last(p2, xs2, dinv, b2r, h1, h2)


def kernel(x, edge_index, W0, b0, g0, be0, W1, b1, g1, be1, W2, b2, g2, be2):
  return _run(x, edge_index, W0, b0, g0, be0, W1, b1, g1, be1, W2, b2, g2,
              be2)


# speedup vs baseline: 1.0794x; 1.0117x over previous
import functools

import jax
import jax.numpy as jnp
from jax import lax
from jax.experimental import pallas as pl
from jax.experimental.pallas import tpu as pltpu
from jax.experimental.pallas import tpu_sc as plsc

N = 10000
D = 128
NC = 2
NS = 16
NW = NC * NS
CHUNK = 128
GRP = 8

ACC_ROWS = 10112
ACC_PER_TILE = ACC_ROWS // NS
EPS = 1e-5
PAD = -1

_MESH = plsc.VectorSubcoreMesh(core_axis_name="c", subcore_axis_name="s")


def _deg_body(nchunks, dst_hbm, zeros_hbm, ones_hbm, deg_out, idx_v, ones_v,
              stage_v, acc_sh):
  c = lax.axis_index("c")
  s = lax.axis_index("s")
  wid = c * NS + s
  pltpu.sync_copy(dst_hbm.at[wid], idx_v)
  pltpu.sync_copy(ones_hbm, ones_v)
  pltpu.sync_copy(zeros_hbm, stage_v)
  pltpu.sync_copy(stage_v, acc_sh.at[pl.ds(s * ACC_PER_TILE, ACC_PER_TILE)])
  plsc.subcore_barrier()
  for j in range(nchunks):
    pltpu.sync_copy(ones_v, acc_sh.at[plsc.Indices(idx_v.at[j], PAD)],
                    add=True)
  plsc.subcore_barrier()
  pltpu.sync_copy(acc_sh.at[pl.ds(s * ACC_PER_TILE, ACC_PER_TILE)], stage_v)
  pltpu.sync_copy(stage_v, deg_out.at[c, s, 0])


def _spmm_body(etile, nchunks, xs_hbm, ef_hbm, dst_hbm, zeros_hbm, pout_hbm,
               sidx0_v, sidx1_v, didx_v, rows_v, semi, semg0, semg1, semz0,
               semz1, acc_sh):
  sidx = (sidx0_v, sidx1_v)
  ngrp = -(-nchunks // GRP)
  c = lax.axis_index("c")
  s = lax.axis_index("s")
  wid = c * NS + s
  sbase = wid * etile

  semg = (semg0, semg1)
  icp = pltpu.async_copy(ef_hbm.at[pl.ds(sbase, GRP * CHUNK)], sidx[0], semi)
  dcp = pltpu.async_copy(dst_hbm.at[wid], didx_v, semg0)

  abase = s * ACC_PER_TILE
  rem = ACC_PER_TILE % CHUNK
  segs = [(abase + k * CHUNK, CHUNK) for k in range(ACC_PER_TILE // CHUNK)]
  if rem:
    segs.append((abase + ACC_PER_TILE - rem, rem))

  @pl.when(c == 0)
  def _():
    semz = (semz0, semz1)
    r0, ln = segs[0]
    a = pltpu.async_copy(xs_hbm.at[pl.ds(r0, ln)],
                         rows_v.at[0, pl.ds(0, ln)], semz[0])
    for k, (r0, ln) in enumerate(segs):
      if k + 1 < len(segs):
        r1, ln1 = segs[k + 1]
        nxt_a = pltpu.async_copy(xs_hbm.at[pl.ds(r1, ln1)],
                                 rows_v.at[(k + 1) % 2, pl.ds(0, ln1)],
                                 semz[(k + 1) % 2])
      a.wait()
      pltpu.sync_copy(rows_v.at[k % 2, pl.ds(0, ln)],
                      acc_sh.at[pl.ds(r0, ln)])
      if k + 1 < len(segs):
        a = nxt_a

  @pl.when(c != 0)
  def _():
    pltpu.sync_copy(zeros_hbm, rows_v.at[0])
    for r0, ln in segs:
      pltpu.sync_copy(rows_v.at[0, pl.ds(0, ln)], acc_sh.at[pl.ds(r0, ln)])

  icp.wait()
  dcp.wait()

  cp = pltpu.async_copy(
      xs_hbm.at[plsc.Indices(sidx[0].at[pl.ds(0, CHUNK)], PAD)],
      rows_v.at[0], semg[0])
  plsc.subcore_barrier()

  for j in range(nchunks):
    g, i = divmod(j, GRP)
    if i == 0 and g + 1 < ngrp:
      icp = pltpu.async_copy(
          ef_hbm.at[pl.ds(sbase + (g + 1) * GRP * CHUNK, GRP * CHUNK)],
          sidx[(g + 1) % 2], semi)
    if j + 1 < nchunks:
      g1, i1 = divmod(j + 1, GRP)
      if i1 == 0:
        icp.wait()
      nxt = pltpu.async_copy(
          xs_hbm.at[plsc.Indices(sidx[g1 % 2].at[pl.ds(i1 * CHUNK, CHUNK)],
                                 PAD)],
          rows_v.at[(j + 1) % 2], semg[(j + 1) % 2])
    cp.wait()
    pltpu.sync_copy(rows_v.at[j % 2],
                    acc_sh.at[plsc.Indices(didx_v.at[j], PAD)], add=True)
    if j + 1 < nchunks:
      cp = nxt
  plsc.subcore_barrier()

  segs = [(abase + k * CHUNK, CHUNK) for k in range(ACC_PER_TILE // CHUNK)]
  if rem:
    segs.append((abase + ACC_PER_TILE - rem, rem))
  pend = [None, None]
  r0, ln = segs[0]
  pltpu.sync_copy(acc_sh.at[pl.ds(r0, ln)], rows_v.at[0, pl.ds(0, ln)])
  for k, (r0, ln) in enumerate(segs):
    b = k % 2
    pend[b] = pltpu.async_copy(rows_v.at[b, pl.ds(0, ln)],
                               pout_hbm.at[c, pl.ds(r0, ln)], semg[b])
    if k + 1 < len(segs):
      nb = (k + 1) % 2
      if pend[nb] is not None:
        pend[nb].wait()
        pend[nb] = None
      r1, ln1 = segs[k + 1]
      pltpu.sync_copy(acc_sh.at[pl.ds(r1, ln1)],
                      rows_v.at[nb, pl.ds(0, ln1)])
  for b in range(2):
    if pend[b] is not None:
      pend[b].wait()


def _make_deg_kernel(ncs):
  return pl.kernel(
      functools.partial(_deg_body, ncs),
      out_type=jax.ShapeDtypeStruct((NC, NS, 1, ACC_PER_TILE), jnp.float32),
      mesh=_MESH,
      scratch_types=[
          pltpu.VMEM((ncs, CHUNK), jnp.int32),
          pltpu.VMEM((CHUNK,), jnp.float32),
          pltpu.VMEM((ACC_PER_TILE,), jnp.float32),
          pltpu.VMEM_SHARED((ACC_ROWS,), jnp.float32),
      ],
  )


def _make_spmm_kernel(etile, ncs, nchunks):
  return pl.kernel(
      functools.partial(_spmm_body, etile, nchunks),
      out_type=jax.ShapeDtypeStruct((NC, ACC_ROWS, D), jnp.float32),
      mesh=_MESH,
      scratch_types=[
          pltpu.VMEM((GRP * CHUNK,), jnp.int32),
          pltpu.VMEM((GRP * CHUNK,), jnp.int32),
          pltpu.VMEM((ncs, CHUNK), jnp.int32),
          pltpu.VMEM((2, CHUNK, D), jnp.float32),
          pltpu.SemaphoreType.DMA,
          pltpu.SemaphoreType.DMA,
          pltpu.SemaphoreType.DMA,
          pltpu.SemaphoreType.DMA,
          pltpu.SemaphoreType.DMA,
          pltpu.VMEM_SHARED((ACC_ROWS, D), jnp.float32),
      ],
  )


def _bn_xw(h, g, be, w, dinv):
  mean = jnp.mean(h, axis=0, keepdims=True)
  dlt = h - mean
  var = jnp.mean(dlt * dlt, axis=0, keepdims=True)
  hb = g * dlt * lax.rsqrt(var + EPS) + be
  return dinv * jnp.dot(hb, w, preferred_element_type=jnp.float32)


def _store_xs(xs_ref, xs):
  xs_ref[:N] = xs
  xs_ref[N:] = jnp.zeros((ACC_ROWS - N, D), jnp.float32)


def _tc_front_body(x_ref, deg_ref, w_ref, g_ref, be_ref, xs_ref, dinv_ref):
  dinv_row = lax.rsqrt(deg_ref[0:1, :] + deg_ref[1:2, :] + 1.0)
  dinv_ref[...] = dinv_row
  dinv = jnp.transpose(dinv_row[:, :N])
  _store_xs(xs_ref, _bn_xw(x_ref[...], g_ref[...], be_ref[...], w_ref[...],
                           dinv))


def _tc_mid_body(p_ref, dinv_ref, b_ref, w_ref, g_ref, be_ref,
                 h_ref, xsn_ref):
  dinv = jnp.transpose(dinv_ref[:, :N])
  ssum = p_ref[0, :N] + p_ref[1, :N]
  h = jnp.maximum(dinv * ssum + b_ref[...], 0.0)
  h_ref[...] = h
  _store_xs(xsn_ref, _bn_xw(h, g_ref[...], be_ref[...], w_ref[...], dinv))


def _tc_last_body(p_ref, dinv_ref, b_ref, h1_ref, h2_ref, out_ref):
  dinv = jnp.transpose(dinv_ref[:, :N])
  ssum = p_ref[0, :N] + p_ref[1, :N]
  h3 = jnp.maximum(dinv * ssum + b_ref[...], 0.0)
  out_ref[...] = jnp.concatenate([h1_ref[...], h2_ref[...], h3], axis=-1)


_tc_front = pl.pallas_call(
    _tc_front_body,
    out_shape=(
        jax.ShapeDtypeStruct((ACC_ROWS, D), jnp.float32),
        jax.ShapeDtypeStruct((1, ACC_ROWS), jnp.float32),
    ),
)

_tc_mid = pl.pallas_call(
    _tc_mid_body,
    out_shape=(
        jax.ShapeDtypeStruct((N, D), jnp.float32),
        jax.ShapeDtypeStruct((ACC_ROWS, D), jnp.float32),
    ),
)

_tc_last = pl.pallas_call(
    _tc_last_body,
    out_shape=jax.ShapeDtypeStruct((N, 3 * D), jnp.float32),
)


@jax.jit
def _run(x, edge_index, W0, b0, g0, be0, W1, b1, g1, be1, W2, b2, g2, be2):
  e = edge_index.shape[1]
  etile = -(-e // (NW * 8)) * 8
  ep = etile * NW
  if ep != e:
    edge_index = jnp.concatenate(
        [edge_index, jnp.full((2, ep - e), PAD, jnp.int32)], axis=1)
  ef = edge_index.reshape(2 * ep)

  nchunks = -(-etile // CHUNK)
  ncs = -(-nchunks // GRP) * GRP
  ppt = ncs * CHUNK - etile
  dflat = lax.slice(ef, (ep,), (2 * ep,)).reshape(NW, etile)
  dst_p = jnp.concatenate(
      [dflat, jnp.full((NW, ppt), PAD, jnp.int32)], axis=1
  ).reshape(NW, ncs, CHUNK)

  zeros_deg = jnp.zeros((ACC_PER_TILE,), jnp.float32)
  zeros_blk = jnp.zeros((CHUNK, D), jnp.float32)
  ones_row = jnp.ones((CHUNK,), jnp.float32)

  deg_p = _make_deg_kernel(ncs)(dst_p, zeros_deg, ones_row)
  deg_p = deg_p.reshape(NC, ACC_ROWS)

  spmm = _make_spmm_kernel(etile, ncs, nchunks)
  g0r, be0r, b0r = g0.reshape(1, D), be0.reshape(1, D), b0.reshape(1, D)
  g1r, be1r, b1r = g1.reshape(1, D), be1.reshape(1, D), b1.reshape(1, D)
  g2r, be2r, b2r = g2.reshape(1, D), be2.reshape(1, D), b2.reshape(1, D)

  xs0, dinv = _tc_front(x, deg_p, W0, g0r, be0r)
  p0 = spmm(xs0, ef, dst_p, zeros_blk)
  h1, xs1 = _tc_mid(p0, dinv, b0r, W1, g1r, be1r)
  p1 = spmm(xs1, ef, dst_p, zeros_blk)
  h2, xs2 = _tc_mid(p1, dinv, b1r, W2, g2r, be2r)
  p2 = spmm(xs2, ef, dst_p, zeros_blk)
  return _tc_last(p2, dinv, b2r, h1, h2)


def kernel(x, edge_index, W0, b0, g0, be0, W1, b1, g1, be1, W2, b2, g2, be2):
  return _run(x, edge_index, W0, b0, g0, be0, W1, b1, g1, be1, W2, b2, g2,
              be2)
